# baseline (speedup 1.0000x reference)
import functools
import jax
import jax.numpy as jnp
from jax import lax
from jax.experimental import pallas as pl
from jax.experimental.pallas import tpu as pltpu
from jax.experimental.pallas import tpu_sc as plsc

_N = 10000
_E = 320000
_D = 128
_ROW_BLK = 400

_L = 16
_H = 64
_NS = 16
_R = 640
_NPAD = _NS * _R
_C = 3200
_NCHUNK = _E // _C
_STAGE = 64
_FBUF = _C + 64
_ACC_ROWS = _R + 1


def _ab_body(x_ref, wtt_ref, wst_ref, bs_ref, a_ref, b_ref):
    xb = x_ref[...]
    b_ref[...] = jnp.dot(xb, wtt_ref[...], preferred_element_type=jnp.float32)
    a_ref[...] = (
        jnp.dot(xb, wst_ref[...], preferred_element_type=jnp.float32)
        + bs_ref[...]
    )


def _compute_ab(x, W_theta, b_theta, W_phi, b_phi):
    wtt = W_theta.T
    wst = (W_theta + W_phi).T
    bs = (b_theta + b_phi).reshape(1, _D)
    grid = _N // _ROW_BLK
    a, b = pl.pallas_call(
        _ab_body,
        grid=(grid,),
        in_specs=[
            pl.BlockSpec((_ROW_BLK, _D), lambda i: (i, 0)),
            pl.BlockSpec((_D, _D), lambda i: (0, 0)),
            pl.BlockSpec((_D, _D), lambda i: (0, 0)),
            pl.BlockSpec((1, _D), lambda i: (0, 0)),
        ],
        out_specs=[
            pl.BlockSpec((_ROW_BLK, _D), lambda i: (i, 0)),
            pl.BlockSpec((_ROW_BLK, _D), lambda i: (i, 0)),
        ],
        out_shape=[
            jax.ShapeDtypeStruct((_N, _D), jnp.float32),
            jax.ShapeDtypeStruct((_N, _D), jnp.float32),
        ],
    )(x, wtt, wst, bs)
    return a, b


def _seg_min_body(b2_hbm, src_hbm, dst_hbm, a2_hbm, out_hbm,
                  acc, srcchA, dstchA, srcchB, dstchB,
                  srcbuf, dstbuf, slab, ring0, ring1, bsh,
                  semSA, semDA, semSB, semDB, sem0, sem1):
    cid = lax.axis_index("c")
    sid = lax.axis_index("s")
    lo = sid * _R
    inf16 = jnp.full((_L,), jnp.inf, jnp.float32)
    iota16 = lax.iota(jnp.int32, _L)

    for k in range(_R // _STAGE):
        woff = (sid * _R + k * _STAGE) * _H
        pltpu.sync_copy(b2_hbm.at[pl.ds(cid * _NPAD * _H + woff,
                                        _STAGE * _H)], slab)
        pltpu.sync_copy(slab, bsh.at[pl.ds(woff, _STAGE * _H)])
    plsc.subcore_barrier()

    def init_body(i, _):
        acc[pl.ds(i * _L, _L)] = inf16
        return 0
    lax.fori_loop(0, (_ACC_ROWS * _H) // _L, init_body, 0)

    def issue_chunk(c, sch, dch, semS, semD):
        pltpu.async_copy(src_hbm.at[pl.ds(c * _C, _C)], sch, semS)
        pltpu.async_copy(dst_hbm.at[pl.ds(c * _C, _C)], dch, semD)

    def wait_chunk(sch, dch, semS, semD):
        pltpu.make_async_copy(src_hbm.at[pl.ds(0, _C)], sch, semS).wait()
        pltpu.make_async_copy(dst_hbm.at[pl.ds(0, _C)], dch, semD).wait()

    def fetch_row(e, ring, sem):
        sv = srcbuf[pl.ds(e, _L)][0]
        pltpu.async_copy(bsh.at[pl.ds(sv * _H, _H)], ring, sem)

    def accumulate(ring, e):
        posv = jnp.full((_L,), e, jnp.int32)
        dvec = plsc.load_gather(dstbuf, [posv])
        base = dvec * _H + iota16
        avs = [plsc.load_gather(acc, [base + j * _L])
               for j in range(_H // _L)]
        rvs = [ring[pl.ds(j * _L, _L)] for j in range(_H // _L)]
        for j in range(_H // _L):
            plsc.store_scatter(acc, [base + j * _L],
                               jnp.minimum(avs[j], rvs[j]))

    def process_chunk(sch, dch):
        def filt_body(i, cnt_vec):
            for h in range(2):
                off = i * 2 * _L + h * _L
                sv = sch[pl.ds(off, _L)]
                dv = dch[pl.ds(off, _L)]
                m = (dv >= lo) & (dv < lo + _R)
                pc = jnp.cumsum(m.astype(jnp.int32))
                pos = cnt_vec + pc - 1
                plsc.store_scatter(srcbuf, [pos], sv, mask=m)
                plsc.store_scatter(dstbuf, [pos], dv - lo, mask=m)
                cnt_vec = cnt_vec + plsc.all_reduce_population_count(m)
            return cnt_vec
        cnt_vec = lax.fori_loop(0, _C // (2 * _L), filt_body,
                                jnp.zeros((_L,), jnp.int32))

        for t in range(2):
            tpos = cnt_vec + t * _L + iota16
            plsc.store_scatter(srcbuf, [tpos], jnp.zeros((_L,), jnp.int32))
            plsc.store_scatter(dstbuf, [tpos],
                               jnp.full((_L,), _R, jnp.int32))

        cnt = jnp.max(cnt_vec)

        fetch_row(0, ring0, sem0)
        fetch_row(1, ring1, sem1)

        def edge_loop(e, _):
            @pl.when(e % 2 == 0)
            def _():
                pltpu.make_async_copy(bsh.at[pl.ds(0, _H)], ring0,
                                      sem0).wait()
                accumulate(ring0, e)
                @pl.when(e < cnt)
                def _():
                    fetch_row(e + 2, ring0, sem0)

            @pl.when(e % 2 == 1)
            def _():
                pltpu.make_async_copy(bsh.at[pl.ds(0, _H)], ring1,
                                      sem1).wait()
                accumulate(ring1, e)
                @pl.when(e < cnt)
                def _():
                    fetch_row(e + 2, ring1, sem1)
            return 0
        lax.fori_loop(0, cnt + 2, edge_loop, 0)

    issue_chunk(0, srcchA, dstchA, semSA, semDA)

    def outer_body(cc, _):
        wait_chunk(srcchA, dstchA, semSA, semDA)
        issue_chunk(2 * cc + 1, srcchB, dstchB, semSB, semDB)
        process_chunk(srcchA, dstchA)

        wait_chunk(srcchB, dstchB, semSB, semDB)
        @pl.when(cc + 1 < _NCHUNK // 2)
        def _():
            issue_chunk(2 * cc + 2, srcchA, dstchA, semSA, semDA)
        process_chunk(srcchB, dstchB)
        return 0
    lax.fori_loop(0, _NCHUNK // 2, outer_body, 0)

    slab_rows = 64
    for s in range(_R // slab_rows):
        woff = (cid * _NPAD + lo + s * slab_rows) * _H
        pltpu.sync_copy(a2_hbm.at[pl.ds(woff, slab_rows * _H)], slab)

        def out_body(i, _):
            av = acc[pl.ds(s * slab_rows * _H + i * _L, _L)]
            sl = slab[pl.ds(i * _L, _L)]
            slab[pl.ds(i * _L, _L)] = jnp.where(av == jnp.inf, 0.0, sl - av)
            return 0
        lax.fori_loop(0, (slab_rows * _H) // _L, out_body, 0)
        pltpu.sync_copy(slab, out_hbm.at[pl.ds(woff, slab_rows * _H)])


@functools.partial(
    pl.kernel,
    mesh=plsc.VectorSubcoreMesh(core_axis_name="c", subcore_axis_name="s"),
    compiler_params=pltpu.CompilerParams(needs_layout_passes=False),
    out_type=jax.ShapeDtypeStruct((2 * _NPAD * _H,), jnp.float32),
    scratch_types=[
        pltpu.VMEM((_ACC_ROWS * _H,), jnp.float32),
        pltpu.VMEM((_C,), jnp.int32),
        pltpu.VMEM((_C,), jnp.int32),
        pltpu.VMEM((_C,), jnp.int32),
        pltpu.VMEM((_C,), jnp.int32),
        pltpu.VMEM((_FBUF,), jnp.int32),
        pltpu.VMEM((_FBUF,), jnp.int32),
        pltpu.VMEM((64 * _H,), jnp.float32),
        pltpu.VMEM((_H,), jnp.float32),
        pltpu.VMEM((_H,), jnp.float32),
        pltpu.VMEM_SHARED((_NPAD * _H,), jnp.float32),
        pltpu.SemaphoreType.DMA,
        pltpu.SemaphoreType.DMA,
        pltpu.SemaphoreType.DMA,
        pltpu.SemaphoreType.DMA,
        pltpu.SemaphoreType.DMA,
        pltpu.SemaphoreType.DMA,
    ],
)
def _seg_min(b2_hbm, src_hbm, dst_hbm, a2_hbm, out_hbm, *scratch):
    _seg_min_body(b2_hbm, src_hbm, dst_hbm, a2_hbm, out_hbm, *scratch)


def kernel(x, edge_index, W_theta, b_theta, W_phi, b_phi):
    a, b = _compute_ab(x, W_theta, b_theta, W_phi, b_phi)
    b_pad = jnp.pad(b, ((0, _NPAD - _N), (0, 0)))
    b2 = jnp.concatenate([b_pad[:, :_H], b_pad[:, _H:]], axis=0).reshape(-1)
    a_pad = jnp.pad(a, ((0, _NPAD - _N), (0, 0)))
    a2 = jnp.concatenate([a_pad[:, :_H], a_pad[:, _H:]], axis=0).reshape(-1)
    src = edge_index[0]
    dst = edge_index[1]
    out_flat = _seg_min(b2, src, dst, a2)
    o3 = out_flat.reshape(2, _NPAD, _H)
    return jnp.concatenate([o3[0, :_N], o3[1, :_N]], axis=1)


# speedup vs baseline: 1.0816x; 1.0816x over previous
import functools
import jax
import jax.numpy as jnp
from jax import lax
from jax.experimental import pallas as pl
from jax.experimental.pallas import tpu as pltpu
from jax.experimental.pallas import tpu_sc as plsc

_N = 10000
_E = 320000
_D = 128
_ROW_BLK = 400

_L = 16
_H = 64
_NS = 16
_R = 640
_NPAD = _NS * _R
_C = 3200
_NCHUNK = _E // _C
_STAGE = 64
_FBUF = _C + 64
_ACC_ROWS = _R + 1


def _ab_body(x_ref, wtt_ref, wst_ref, bs_ref, a_ref, b_ref):
    xb = x_ref[...]
    b_ref[...] = jnp.dot(xb, wtt_ref[...], preferred_element_type=jnp.float32)
    a_ref[...] = (
        jnp.dot(xb, wst_ref[...], preferred_element_type=jnp.float32)
        + bs_ref[...]
    )


def _compute_ab(x, W_theta, b_theta, W_phi, b_phi):
    wtt = W_theta.T
    wst = (W_theta + W_phi).T
    bs = (b_theta + b_phi).reshape(1, _D)
    grid = _N // _ROW_BLK
    a, b = pl.pallas_call(
        _ab_body,
        grid=(grid,),
        in_specs=[
            pl.BlockSpec((_ROW_BLK, _D), lambda i: (i, 0)),
            pl.BlockSpec((_D, _D), lambda i: (0, 0)),
            pl.BlockSpec((_D, _D), lambda i: (0, 0)),
            pl.BlockSpec((1, _D), lambda i: (0, 0)),
        ],
        out_specs=[
            pl.BlockSpec((_ROW_BLK, _D), lambda i: (i, 0)),
            pl.BlockSpec((_ROW_BLK, _D), lambda i: (i, 0)),
        ],
        out_shape=[
            jax.ShapeDtypeStruct((_N, _D), jnp.float32),
            jax.ShapeDtypeStruct((_N, _D), jnp.float32),
        ],
    )(x, wtt, wst, bs)
    return a, b


def _seg_min_body(b2_hbm, src_hbm, dst_hbm, a2_hbm, out_hbm,
                  acc, srcchA, dstchA, srcchB, dstchB,
                  srcbuf, dstbuf, slab, ring0, ring1, bsh,
                  semSA, semDA, semSB, semDB, sem0, sem1):
    cid = lax.axis_index("c")
    sid = lax.axis_index("s")
    lo = sid * _R
    inf16 = jnp.full((_L,), jnp.inf, jnp.float32)
    iota16 = lax.iota(jnp.int32, _L)

    for k in range(_R // _STAGE):
        woff = (sid * _R + k * _STAGE) * _H
        pltpu.sync_copy(b2_hbm.at[pl.ds(cid * _NPAD * _H + woff,
                                        _STAGE * _H)], slab)
        pltpu.sync_copy(slab, bsh.at[pl.ds(woff, _STAGE * _H)])
    plsc.subcore_barrier()

    def init_body(i, _):
        acc[pl.ds(i * _L, _L)] = inf16
        return 0
    lax.fori_loop(0, (_ACC_ROWS * _H) // _L, init_body, 0)

    def issue_chunk(c, sch, dch, semS, semD):
        pltpu.async_copy(src_hbm.at[pl.ds(c * _C, _C)], sch, semS)
        pltpu.async_copy(dst_hbm.at[pl.ds(c * _C, _C)], dch, semD)

    def wait_chunk(sch, dch, semS, semD):
        pltpu.make_async_copy(src_hbm.at[pl.ds(0, _C)], sch, semS).wait()
        pltpu.make_async_copy(dst_hbm.at[pl.ds(0, _C)], dch, semD).wait()

    def fetch_row(sv, ring, sem):
        pltpu.async_copy(bsh.at[pl.ds(sv * _H, _H)], ring, sem)

    def accumulate(ring, e):
        posv = jnp.full((_L,), e, jnp.int32)
        dvec = plsc.load_gather(dstbuf, [posv])
        base = dvec * _H + iota16
        avs = [plsc.load_gather(acc, [base + j * _L])
               for j in range(_H // _L)]
        rvs = [ring[pl.ds(j * _L, _L)] for j in range(_H // _L)]
        for j in range(_H // _L):
            plsc.store_scatter(acc, [base + j * _L],
                               jnp.minimum(avs[j], rvs[j]))

    def process_chunk(sch, dch):
        def filt_body(i, cnt_vec):
            for h in range(2):
                off = i * 2 * _L + h * _L
                sv = sch[pl.ds(off, _L)]
                dv = dch[pl.ds(off, _L)]
                m = (dv >= lo) & (dv < lo + _R)
                pc = jnp.cumsum(m.astype(jnp.int32))
                pos = cnt_vec + pc - 1
                plsc.store_scatter(srcbuf, [pos], sv, mask=m)
                plsc.store_scatter(dstbuf, [pos], dv - lo, mask=m)
                cnt_vec = cnt_vec + plsc.all_reduce_population_count(m)
            return cnt_vec
        cnt_vec = lax.fori_loop(0, _C // (2 * _L), filt_body,
                                jnp.zeros((_L,), jnp.int32))

        for t in range(2):
            tpos = cnt_vec + t * _L + iota16
            plsc.store_scatter(srcbuf, [tpos], jnp.zeros((_L,), jnp.int32))
            plsc.store_scatter(dstbuf, [tpos],
                               jnp.full((_L,), _R, jnp.int32))

        cnt = jnp.max(cnt_vec)
        tot = 2 * ((cnt + 3) // 2)

        sv01 = srcbuf[pl.ds(0, _L)]
        fetch_row(sv01[0], ring0, sem0)
        fetch_row(sv01[1], ring1, sem1)

        def pair_loop(p, _):
            e = 2 * p
            svn = srcbuf[pl.ds(e + 2, _L)]
            sv0 = svn[0]
            sv1 = svn[1]
            pltpu.make_async_copy(bsh.at[pl.ds(0, _H)], ring0, sem0).wait()
            accumulate(ring0, e)
            @pl.when(e + 2 < tot)
            def _():
                fetch_row(sv0, ring0, sem0)
            pltpu.make_async_copy(bsh.at[pl.ds(0, _H)], ring1, sem1).wait()
            accumulate(ring1, e + 1)
            @pl.when(e + 3 < tot)
            def _():
                fetch_row(sv1, ring1, sem1)
            return 0
        lax.fori_loop(0, tot // 2, pair_loop, 0)

    issue_chunk(0, srcchA, dstchA, semSA, semDA)

    def outer_body(cc, _):
        wait_chunk(srcchA, dstchA, semSA, semDA)
        issue_chunk(2 * cc + 1, srcchB, dstchB, semSB, semDB)
        process_chunk(srcchA, dstchA)

        wait_chunk(srcchB, dstchB, semSB, semDB)
        @pl.when(cc + 1 < _NCHUNK // 2)
        def _():
            issue_chunk(2 * cc + 2, srcchA, dstchA, semSA, semDA)
        process_chunk(srcchB, dstchB)
        return 0
    lax.fori_loop(0, _NCHUNK // 2, outer_body, 0)

    slab_rows = 64
    for s in range(_R // slab_rows):
        woff = (cid * _NPAD + lo + s * slab_rows) * _H
        pltpu.sync_copy(a2_hbm.at[pl.ds(woff, slab_rows * _H)], slab)

        def out_body(i, _):
            av = acc[pl.ds(s * slab_rows * _H + i * _L, _L)]
            sl = slab[pl.ds(i * _L, _L)]
            slab[pl.ds(i * _L, _L)] = jnp.where(av == jnp.inf, 0.0, sl - av)
            return 0
        lax.fori_loop(0, (slab_rows * _H) // _L, out_body, 0)
        pltpu.sync_copy(slab, out_hbm.at[pl.ds(woff, slab_rows * _H)])


@functools.partial(
    pl.kernel,
    mesh=plsc.VectorSubcoreMesh(core_axis_name="c", subcore_axis_name="s"),
    compiler_params=pltpu.CompilerParams(needs_layout_passes=False),
    out_type=jax.ShapeDtypeStruct((2 * _NPAD * _H,), jnp.float32),
    scratch_types=[
        pltpu.VMEM((_ACC_ROWS * _H,), jnp.float32),
        pltpu.VMEM((_C,), jnp.int32),
        pltpu.VMEM((_C,), jnp.int32),
        pltpu.VMEM((_C,), jnp.int32),
        pltpu.VMEM((_C,), jnp.int32),
        pltpu.VMEM((_FBUF,), jnp.int32),
        pltpu.VMEM((_FBUF,), jnp.int32),
        pltpu.VMEM((64 * _H,), jnp.float32),
        pltpu.VMEM((_H,), jnp.float32),
        pltpu.VMEM((_H,), jnp.float32),
        pltpu.VMEM_SHARED((_NPAD * _H,), jnp.float32),
        pltpu.SemaphoreType.DMA,
        pltpu.SemaphoreType.DMA,
        pltpu.SemaphoreType.DMA,
        pltpu.SemaphoreType.DMA,
        pltpu.SemaphoreType.DMA,
        pltpu.SemaphoreType.DMA,
    ],
)
def _seg_min(b2_hbm, src_hbm, dst_hbm, a2_hbm, out_hbm, *scratch):
    _seg_min_body(b2_hbm, src_hbm, dst_hbm, a2_hbm, out_hbm, *scratch)


def kernel(x, edge_index, W_theta, b_theta, W_phi, b_phi):
    a, b = _compute_ab(x, W_theta, b_theta, W_phi, b_phi)
    b_pad = jnp.pad(b, ((0, _NPAD - _N), (0, 0)))
    b2 = jnp.concatenate([b_pad[:, :_H], b_pad[:, _H:]], axis=0).reshape(-1)
    a_pad = jnp.pad(a, ((0, _NPAD - _N), (0, 0)))
    a2 = jnp.concatenate([a_pad[:, :_H], a_pad[:, _H:]], axis=0).reshape(-1)
    src = edge_index[0]
    dst = edge_index[1]
    out_flat = _seg_min(b2, src, dst, a2)
    o3 = out_flat.reshape(2, _NPAD, _H)
    return jnp.concatenate([o3[0, :_N], o3[1, :_N]], axis=1)


# speedup vs baseline: 1.6529x; 1.5283x over previous
import functools
import jax
import jax.numpy as jnp
from jax import lax
from jax.experimental import pallas as pl
from jax.experimental.pallas import tpu as pltpu
from jax.experimental.pallas import tpu_sc as plsc

_N = 10000
_E = 320000
_D = 128
_ROW_BLK = 400

_L = 16
_H = 64
_NS = 16
_R = 640
_NPAD = _NS * _R
_C = 3200
_NCHUNK = _E // _C
_STAGE = 64
_FBUF = _C + 64
_ACC_ROWS = _R + 1


def _ab_body(x_ref, wtt_ref, wst_ref, bs_ref, a_ref, b_ref):
    xb = x_ref[...]
    b_ref[...] = jnp.dot(xb, wtt_ref[...], preferred_element_type=jnp.float32)
    a_ref[...] = (
        jnp.dot(xb, wst_ref[...], preferred_element_type=jnp.float32)
        + bs_ref[...]
    )


def _compute_ab(x, W_theta, b_theta, W_phi, b_phi):
    wtt = W_theta.T
    wst = (W_theta + W_phi).T
    bs = (b_theta + b_phi).reshape(1, _D)
    grid = _N // _ROW_BLK
    a, b = pl.pallas_call(
        _ab_body,
        grid=(grid,),
        in_specs=[
            pl.BlockSpec((_ROW_BLK, _D), lambda i: (i, 0)),
            pl.BlockSpec((_D, _D), lambda i: (0, 0)),
            pl.BlockSpec((_D, _D), lambda i: (0, 0)),
            pl.BlockSpec((1, _D), lambda i: (0, 0)),
        ],
        out_specs=[
            pl.BlockSpec((_ROW_BLK, _D), lambda i: (i, 0)),
            pl.BlockSpec((_ROW_BLK, _D), lambda i: (i, 0)),
        ],
        out_shape=[
            jax.ShapeDtypeStruct((_N, _D), jnp.float32),
            jax.ShapeDtypeStruct((_N, _D), jnp.float32),
        ],
    )(x, wtt, wst, bs)
    return a, b


def _seg_min_body(b2_hbm, src_hbm, dst_hbm, a2_hbm, out_hbm,
                  acc, srcchA, dstchA, srcchB, dstchB,
                  srcbuf, dstbuf, slab, ring0, ring1, ring2, ring3, bsh,
                  semSA, semDA, semSB, semDB, sem0, sem1, sem2, sem3):
    rings = [ring0, ring1, ring2, ring3]
    sems = [sem0, sem1, sem2, sem3]
    cid = lax.axis_index("c")
    sid = lax.axis_index("s")
    lo = sid * _R
    inf16 = jnp.full((_L,), jnp.inf, jnp.float32)
    iota16 = lax.iota(jnp.int32, _L)

    for k in range(_R // _STAGE):
        woff = (sid * _R + k * _STAGE) * _H
        pltpu.sync_copy(b2_hbm.at[pl.ds(cid * _NPAD * _H + woff,
                                        _STAGE * _H)], slab)
        pltpu.sync_copy(slab, bsh.at[pl.ds(woff, _STAGE * _H)])
    plsc.subcore_barrier()

    def init_body(i, _):
        acc[pl.ds(i * _L, _L)] = inf16
        return 0
    lax.fori_loop(0, (_ACC_ROWS * _H) // _L, init_body, 0)

    def issue_chunk(c, sch, dch, semS, semD):
        pltpu.async_copy(src_hbm.at[pl.ds(c * _C, _C)], sch, semS)
        pltpu.async_copy(dst_hbm.at[pl.ds(c * _C, _C)], dch, semD)

    def wait_chunk(sch, dch, semS, semD):
        pltpu.make_async_copy(src_hbm.at[pl.ds(0, _C)], sch, semS).wait()
        pltpu.make_async_copy(dst_hbm.at[pl.ds(0, _C)], dch, semD).wait()

    def fetch_row(sv, ring, sem):
        pltpu.async_copy(bsh.at[pl.ds(sv * _H, _H)], ring, sem)

    def accumulate(ring, e):
        posv = jnp.full((_L,), e, jnp.int32)
        dvec = plsc.load_gather(dstbuf, [posv])
        base = dvec * _H + iota16
        avs = [plsc.load_gather(acc, [base + j * _L])
               for j in range(_H // _L)]
        rvs = [ring[pl.ds(j * _L, _L)] for j in range(_H // _L)]
        for j in range(_H // _L):
            plsc.store_scatter(acc, [base + j * _L],
                               jnp.minimum(avs[j], rvs[j]))

    def process_chunk(sch, dch):
        def filt_body(i, cnt_vec):
            for h in range(2):
                off = i * 2 * _L + h * _L
                sv = sch[pl.ds(off, _L)]
                dv = dch[pl.ds(off, _L)]
                m = (dv >= lo) & (dv < lo + _R)
                pc = jnp.cumsum(m.astype(jnp.int32))
                pos = cnt_vec + pc - 1
                plsc.store_scatter(srcbuf, [pos], sv, mask=m)
                plsc.store_scatter(dstbuf, [pos], dv - lo, mask=m)
                cnt_vec = cnt_vec + plsc.all_reduce_population_count(m)
            return cnt_vec
        cnt_vec = lax.fori_loop(0, _C // (2 * _L), filt_body,
                                jnp.zeros((_L,), jnp.int32))

        for t in range(2):
            tpos = cnt_vec + t * _L + iota16
            plsc.store_scatter(srcbuf, [tpos], jnp.zeros((_L,), jnp.int32))
            plsc.store_scatter(dstbuf, [tpos],
                               jnp.full((_L,), _R, jnp.int32))

        cnt = jnp.max(cnt_vec)
        tot = 4 * ((cnt + 7) // 4)

        sv0_ = srcbuf[pl.ds(0, _L)]
        for i in range(4):
            fetch_row(sv0_[i], rings[i], sems[i])

        def quad_loop(q, _):
            e = 4 * q
            svn = srcbuf[pl.ds(e + 4, _L)]
            svs = [svn[i] for i in range(4)]
            for i in range(4):
                pltpu.make_async_copy(bsh.at[pl.ds(0, _H)], rings[i],
                                      sems[i]).wait()
                accumulate(rings[i], e + i)
                @pl.when(e + 4 + i < tot)
                def _(i=i):
                    fetch_row(svs[i], rings[i], sems[i])
            return 0
        lax.fori_loop(0, tot // 4, quad_loop, 0)

    issue_chunk(0, srcchA, dstchA, semSA, semDA)

    def outer_body(cc, _):
        wait_chunk(srcchA, dstchA, semSA, semDA)
        issue_chunk(2 * cc + 1, srcchB, dstchB, semSB, semDB)
        process_chunk(srcchA, dstchA)

        wait_chunk(srcchB, dstchB, semSB, semDB)
        @pl.when(cc + 1 < _NCHUNK // 2)
        def _():
            issue_chunk(2 * cc + 2, srcchA, dstchA, semSA, semDA)
        process_chunk(srcchB, dstchB)
        return 0
    lax.fori_loop(0, _NCHUNK // 2, outer_body, 0)

    slab_rows = 64
    for s in range(_R // slab_rows):
        woff = (cid * _NPAD + lo + s * slab_rows) * _H
        pltpu.sync_copy(a2_hbm.at[pl.ds(woff, slab_rows * _H)], slab)

        def out_body(i, _):
            av = acc[pl.ds(s * slab_rows * _H + i * _L, _L)]
            sl = slab[pl.ds(i * _L, _L)]
            slab[pl.ds(i * _L, _L)] = jnp.where(av == jnp.inf, 0.0, sl - av)
            return 0
        lax.fori_loop(0, (slab_rows * _H) // _L, out_body, 0)
        pltpu.sync_copy(slab, out_hbm.at[pl.ds(woff, slab_rows * _H)])


@functools.partial(
    pl.kernel,
    mesh=plsc.VectorSubcoreMesh(core_axis_name="c", subcore_axis_name="s"),
    compiler_params=pltpu.CompilerParams(needs_layout_passes=False),
    out_type=jax.ShapeDtypeStruct((2 * _NPAD * _H,), jnp.float32),
    scratch_types=[
        pltpu.VMEM((_ACC_ROWS * _H,), jnp.float32),
        pltpu.VMEM((_C,), jnp.int32),
        pltpu.VMEM((_C,), jnp.int32),
        pltpu.VMEM((_C,), jnp.int32),
        pltpu.VMEM((_C,), jnp.int32),
        pltpu.VMEM((_FBUF,), jnp.int32),
        pltpu.VMEM((_FBUF,), jnp.int32),
        pltpu.VMEM((64 * _H,), jnp.float32),
        pltpu.VMEM((_H,), jnp.float32),
        pltpu.VMEM((_H,), jnp.float32),
        pltpu.VMEM((_H,), jnp.float32),
        pltpu.VMEM((_H,), jnp.float32),
        pltpu.VMEM_SHARED((_NPAD * _H,), jnp.float32),
        pltpu.SemaphoreType.DMA,
        pltpu.SemaphoreType.DMA,
        pltpu.SemaphoreType.DMA,
        pltpu.SemaphoreType.DMA,
        pltpu.SemaphoreType.DMA,
        pltpu.SemaphoreType.DMA,
        pltpu.SemaphoreType.DMA,
        pltpu.SemaphoreType.DMA,
    ],
)
def _seg_min(b2_hbm, src_hbm, dst_hbm, a2_hbm, out_hbm, *scratch):
    _seg_min_body(b2_hbm, src_hbm, dst_hbm, a2_hbm, out_hbm, *scratch)


def kernel(x, edge_index, W_theta, b_theta, W_phi, b_phi):
    a, b = _compute_ab(x, W_theta, b_theta, W_phi, b_phi)
    b_pad = jnp.pad(b, ((0, _NPAD - _N), (0, 0)))
    b2 = jnp.concatenate([b_pad[:, :_H], b_pad[:, _H:]], axis=0).reshape(-1)
    a_pad = jnp.pad(a, ((0, _NPAD - _N), (0, 0)))
    a2 = jnp.concatenate([a_pad[:, :_H], a_pad[:, _H:]], axis=0).reshape(-1)
    src = edge_index[0]
    dst = edge_index[1]
    out_flat = _seg_min(b2, src, dst, a2)
    o3 = out_flat.reshape(2, _NPAD, _H)
    return jnp.concatenate([o3[0, :_N], o3[1, :_N]], axis=1)


# speedup vs baseline: 2.0038x; 1.2122x over previous
import functools
import jax
import jax.numpy as jnp
from jax import lax
from jax.experimental import pallas as pl
from jax.experimental.pallas import tpu as pltpu
from jax.experimental.pallas import tpu_sc as plsc

_N = 10000
_E = 320000
_D = 128
_ROW_BLK = 400

_L = 16
_H = 64
_NS = 16
_R = 640
_NPAD = _NS * _R
_C = 3200
_NCHUNK = _E // _C
_STAGE = 64
_FBUF = _C + 64
_ACC_ROWS = _R + 1


def _ab_body(x_ref, wtt_ref, wst_ref, bs_ref, a_ref, b_ref):
    xb = x_ref[...]
    b_ref[...] = jnp.dot(xb, wtt_ref[...], preferred_element_type=jnp.float32)
    a_ref[...] = (
        jnp.dot(xb, wst_ref[...], preferred_element_type=jnp.float32)
        + bs_ref[...]
    )


def _compute_ab(x, W_theta, b_theta, W_phi, b_phi):
    wtt = W_theta.T
    wst = (W_theta + W_phi).T
    bs = (b_theta + b_phi).reshape(1, _D)
    grid = _N // _ROW_BLK
    a, b = pl.pallas_call(
        _ab_body,
        grid=(grid,),
        in_specs=[
            pl.BlockSpec((_ROW_BLK, _D), lambda i: (i, 0)),
            pl.BlockSpec((_D, _D), lambda i: (0, 0)),
            pl.BlockSpec((_D, _D), lambda i: (0, 0)),
            pl.BlockSpec((1, _D), lambda i: (0, 0)),
        ],
        out_specs=[
            pl.BlockSpec((_ROW_BLK, _D), lambda i: (i, 0)),
            pl.BlockSpec((_ROW_BLK, _D), lambda i: (i, 0)),
        ],
        out_shape=[
            jax.ShapeDtypeStruct((_N, _D), jnp.float32),
            jax.ShapeDtypeStruct((_N, _D), jnp.float32),
        ],
    )(x, wtt, wst, bs)
    return a, b


def _seg_min_body(b2_hbm, src_hbm, dst_hbm, a2_hbm, out_hbm,
                  acc, srcchA, dstchA, srcchB, dstchB,
                  srcbuf, dstbuf, slab,
                  ring0, ring1, ring2, ring3, ring4, ring5, ring6, ring7,
                  bsh, semSA, semDA, semSB, semDB,
                  sem0, sem1, sem2, sem3, sem4, sem5, sem6, sem7):
    rings = [ring0, ring1, ring2, ring3, ring4, ring5, ring6, ring7]
    sems = [sem0, sem1, sem2, sem3, sem4, sem5, sem6, sem7]
    cid = lax.axis_index("c")
    sid = lax.axis_index("s")
    lo = sid * _R
    inf16 = jnp.full((_L,), jnp.inf, jnp.float32)
    iota16 = lax.iota(jnp.int32, _L)

    for k in range(_R // _STAGE):
        woff = (sid * _R + k * _STAGE) * _H
        pltpu.sync_copy(b2_hbm.at[pl.ds(cid * _NPAD * _H + woff,
                                        _STAGE * _H)], slab)
        pltpu.sync_copy(slab, bsh.at[pl.ds(woff, _STAGE * _H)])
    plsc.subcore_barrier()

    def init_body(i, _):
        acc[pl.ds(i * _L, _L)] = inf16
        return 0
    lax.fori_loop(0, (_ACC_ROWS * _H) // _L, init_body, 0)

    def issue_chunk(c, sch, dch, semS, semD):
        pltpu.async_copy(src_hbm.at[pl.ds(c * _C, _C)], sch, semS)
        pltpu.async_copy(dst_hbm.at[pl.ds(c * _C, _C)], dch, semD)

    def wait_chunk(sch, dch, semS, semD):
        pltpu.make_async_copy(src_hbm.at[pl.ds(0, _C)], sch, semS).wait()
        pltpu.make_async_copy(dst_hbm.at[pl.ds(0, _C)], dch, semD).wait()

    def fetch_row(sv, ring, sem):
        pltpu.async_copy(bsh.at[pl.ds(sv * _H, _H)], ring, sem)

    def accumulate(ring, e):
        posv = jnp.full((_L,), e, jnp.int32)
        dvec = plsc.load_gather(dstbuf, [posv])
        base = dvec * _H + iota16
        avs = [plsc.load_gather(acc, [base + j * _L])
               for j in range(_H // _L)]
        rvs = [ring[pl.ds(j * _L, _L)] for j in range(_H // _L)]
        for j in range(_H // _L):
            plsc.store_scatter(acc, [base + j * _L],
                               jnp.minimum(avs[j], rvs[j]))

    def process_chunk(sch, dch):
        def filt_body(i, cnt_vec):
            for h in range(2):
                off = i * 2 * _L + h * _L
                sv = sch[pl.ds(off, _L)]
                dv = dch[pl.ds(off, _L)]
                m = (dv >= lo) & (dv < lo + _R)
                pc = jnp.cumsum(m.astype(jnp.int32))
                pos = cnt_vec + pc - 1
                plsc.store_scatter(srcbuf, [pos], sv, mask=m)
                plsc.store_scatter(dstbuf, [pos], dv - lo, mask=m)
                cnt_vec = cnt_vec + plsc.all_reduce_population_count(m)
            return cnt_vec
        cnt_vec = lax.fori_loop(0, _C // (2 * _L), filt_body,
                                jnp.zeros((_L,), jnp.int32))

        for t in range(2):
            tpos = cnt_vec + t * _L + iota16
            plsc.store_scatter(srcbuf, [tpos], jnp.zeros((_L,), jnp.int32))
            plsc.store_scatter(dstbuf, [tpos],
                               jnp.full((_L,), _R, jnp.int32))

        cnt = jnp.max(cnt_vec)
        tot = 8 * ((cnt + 15) // 8)

        sv0_ = srcbuf[pl.ds(0, _L)]
        for i in range(8):
            fetch_row(sv0_[i], rings[i], sems[i])

        def oct_loop(q, _):
            e = 8 * q
            svn = srcbuf[pl.ds(e + 8, _L)]
            svs = [svn[i] for i in range(8)]
            for i in range(8):
                pltpu.make_async_copy(bsh.at[pl.ds(0, _H)], rings[i],
                                      sems[i]).wait()
                accumulate(rings[i], e + i)
                @pl.when(e + 8 + i < tot)
                def _(i=i):
                    fetch_row(svs[i], rings[i], sems[i])
            return 0
        lax.fori_loop(0, tot // 8, oct_loop, 0)

    issue_chunk(0, srcchA, dstchA, semSA, semDA)

    def outer_body(cc, _):
        wait_chunk(srcchA, dstchA, semSA, semDA)
        issue_chunk(2 * cc + 1, srcchB, dstchB, semSB, semDB)
        process_chunk(srcchA, dstchA)

        wait_chunk(srcchB, dstchB, semSB, semDB)
        @pl.when(cc + 1 < _NCHUNK // 2)
        def _():
            issue_chunk(2 * cc + 2, srcchA, dstchA, semSA, semDA)
        process_chunk(srcchB, dstchB)
        return 0
    lax.fori_loop(0, _NCHUNK // 2, outer_body, 0)

    slab_rows = 64
    for s in range(_R // slab_rows):
        woff = (cid * _NPAD + lo + s * slab_rows) * _H
        pltpu.sync_copy(a2_hbm.at[pl.ds(woff, slab_rows * _H)], slab)

        def out_body(i, _):
            av = acc[pl.ds(s * slab_rows * _H + i * _L, _L)]
            sl = slab[pl.ds(i * _L, _L)]
            slab[pl.ds(i * _L, _L)] = jnp.where(av == jnp.inf, 0.0, sl - av)
            return 0
        lax.fori_loop(0, (slab_rows * _H) // _L, out_body, 0)
        pltpu.sync_copy(slab, out_hbm.at[pl.ds(woff, slab_rows * _H)])


@functools.partial(
    pl.kernel,
    mesh=plsc.VectorSubcoreMesh(core_axis_name="c", subcore_axis_name="s"),
    compiler_params=pltpu.CompilerParams(needs_layout_passes=False),
    out_type=jax.ShapeDtypeStruct((2 * _NPAD * _H,), jnp.float32),
    scratch_types=[
        pltpu.VMEM((_ACC_ROWS * _H,), jnp.float32),
        pltpu.VMEM((_C,), jnp.int32),
        pltpu.VMEM((_C,), jnp.int32),
        pltpu.VMEM((_C,), jnp.int32),
        pltpu.VMEM((_C,), jnp.int32),
        pltpu.VMEM((_FBUF,), jnp.int32),
        pltpu.VMEM((_FBUF,), jnp.int32),
        pltpu.VMEM((64 * _H,), jnp.float32),
        pltpu.VMEM((_H,), jnp.float32),
        pltpu.VMEM((_H,), jnp.float32),
        pltpu.VMEM((_H,), jnp.float32),
        pltpu.VMEM((_H,), jnp.float32),
        pltpu.VMEM((_H,), jnp.float32),
        pltpu.VMEM((_H,), jnp.float32),
        pltpu.VMEM((_H,), jnp.float32),
        pltpu.VMEM((_H,), jnp.float32),
        pltpu.VMEM_SHARED((_NPAD * _H,), jnp.float32),
        pltpu.SemaphoreType.DMA,
        pltpu.SemaphoreType.DMA,
        pltpu.SemaphoreType.DMA,
        pltpu.SemaphoreType.DMA,
        pltpu.SemaphoreType.DMA,
        pltpu.SemaphoreType.DMA,
        pltpu.SemaphoreType.DMA,
        pltpu.SemaphoreType.DMA,
        pltpu.SemaphoreType.DMA,
        pltpu.SemaphoreType.DMA,
        pltpu.SemaphoreType.DMA,
        pltpu.SemaphoreType.DMA,
    ],
)
def _seg_min(b2_hbm, src_hbm, dst_hbm, a2_hbm, out_hbm, *scratch):
    _seg_min_body(b2_hbm, src_hbm, dst_hbm, a2_hbm, out_hbm, *scratch)


def kernel(x, edge_index, W_theta, b_theta, W_phi, b_phi):
    a, b = _compute_ab(x, W_theta, b_theta, W_phi, b_phi)
    b_pad = jnp.pad(b, ((0, _NPAD - _N), (0, 0)))
    b2 = jnp.concatenate([b_pad[:, :_H], b_pad[:, _H:]], axis=0).reshape(-1)
    a_pad = jnp.pad(a, ((0, _NPAD - _N), (0, 0)))
    a2 = jnp.concatenate([a_pad[:, :_H], a_pad[:, _H:]], axis=0).reshape(-1)
    src = edge_index[0]
    dst = edge_index[1]
    out_flat = _seg_min(b2, src, dst, a2)
    o3 = out_flat.reshape(2, _NPAD, _H)
    return jnp.concatenate([o3[0, :_N], o3[1, :_N]], axis=1)


# speedup vs baseline: 2.0118x; 1.0040x over previous
import functools
import jax
import jax.numpy as jnp
from jax import lax
from jax.experimental import pallas as pl
from jax.experimental.pallas import tpu as pltpu
from jax.experimental.pallas import tpu_sc as plsc

_N = 10000
_E = 320000
_D = 128
_ROW_BLK = 400

_L = 16
_H = 64
_NS = 16
_R = 640
_NPAD = _NS * _R
_C = 3200
_NCHUNK = _E // _C
_STAGE = 64
_FBUF = _C + 64
_ACC_ROWS = _R + 1


def _ab_body(x_ref, wtt_ref, wst_ref, bs_ref, a_ref, b_ref):
    xb = x_ref[...]
    b_ref[...] = jnp.dot(xb, wtt_ref[...], preferred_element_type=jnp.float32)
    a_ref[...] = (
        jnp.dot(xb, wst_ref[...], preferred_element_type=jnp.float32)
        + bs_ref[...]
    )


def _compute_ab(x, W_theta, b_theta, W_phi, b_phi):
    wtt = W_theta.T
    wst = (W_theta + W_phi).T
    bs = (b_theta + b_phi).reshape(1, _D)
    grid = _N // _ROW_BLK
    a, b = pl.pallas_call(
        _ab_body,
        grid=(grid,),
        in_specs=[
            pl.BlockSpec((_ROW_BLK, _D), lambda i: (i, 0)),
            pl.BlockSpec((_D, _D), lambda i: (0, 0)),
            pl.BlockSpec((_D, _D), lambda i: (0, 0)),
            pl.BlockSpec((1, _D), lambda i: (0, 0)),
        ],
        out_specs=[
            pl.BlockSpec((_ROW_BLK, _D), lambda i: (i, 0)),
            pl.BlockSpec((_ROW_BLK, _D), lambda i: (i, 0)),
        ],
        out_shape=[
            jax.ShapeDtypeStruct((_N, _D), jnp.float32),
            jax.ShapeDtypeStruct((_N, _D), jnp.float32),
        ],
    )(x, wtt, wst, bs)
    return a, b


def _seg_min_body(b2_hbm, src_hbm, dst_hbm, a2_hbm, out_hbm,
                  acc, srcchA, dstchA, srcchB, dstchB,
                  srcbuf, dstbuf, slab,
                  ring0, ring1, ring2, ring3, ring4, ring5, ring6, ring7,
                  bsh, semSA, semDA, semSB, semDB,
                  sem0, sem1, sem2, sem3, sem4, sem5, sem6, sem7):
    rings = [ring0, ring1, ring2, ring3, ring4, ring5, ring6, ring7]
    sems = [sem0, sem1, sem2, sem3, sem4, sem5, sem6, sem7]
    cid = lax.axis_index("c")
    sid = lax.axis_index("s")
    lo = sid * _R
    inf16 = jnp.full((_L,), jnp.inf, jnp.float32)
    iota16 = lax.iota(jnp.int32, _L)

    for k in range(_R // _STAGE):
        woff = (sid * _R + k * _STAGE) * _H
        pltpu.sync_copy(b2_hbm.at[pl.ds(cid * _NPAD * _H + woff,
                                        _STAGE * _H)], slab)
        pltpu.sync_copy(slab, bsh.at[pl.ds(woff, _STAGE * _H)])
    plsc.subcore_barrier()

    def init_body(i, _):
        acc[pl.ds(i * _L, _L)] = inf16
        return 0
    lax.fori_loop(0, (_ACC_ROWS * _H) // _L, init_body, 0)

    def issue_chunk(c, sch, dch, semS, semD):
        pltpu.async_copy(src_hbm.at[pl.ds(c * _C, _C)], sch, semS)
        pltpu.async_copy(dst_hbm.at[pl.ds(c * _C, _C)], dch, semD)

    def wait_chunk(sch, dch, semS, semD):
        pltpu.make_async_copy(src_hbm.at[pl.ds(0, _C)], sch, semS).wait()
        pltpu.make_async_copy(dst_hbm.at[pl.ds(0, _C)], dch, semD).wait()

    def fetch_row(sv, ring, sem):
        pltpu.async_copy(bsh.at[pl.ds(sv * _H, _H)], ring, sem)

    def accumulate(ring, e):
        posv = jnp.full((_L,), e, jnp.int32)
        dvec = plsc.load_gather(dstbuf, [posv])
        base = dvec * _H + iota16
        avs = [plsc.load_gather(acc, [base + j * _L])
               for j in range(_H // _L)]
        rvs = [ring[pl.ds(j * _L, _L)] for j in range(_H // _L)]
        for j in range(_H // _L):
            plsc.store_scatter(acc, [base + j * _L],
                               jnp.minimum(avs[j], rvs[j]))

    def process_chunk(sch, dch):
        def filt_body(i, cnt_vec):
            for h in range(4):
                off = i * 4 * _L + h * _L
                sv = sch[pl.ds(off, _L)]
                dv = dch[pl.ds(off, _L)]
                m = (dv >= lo) & (dv < lo + _R)
                pc = jnp.cumsum(m.astype(jnp.int32))
                pos = cnt_vec + pc - 1
                plsc.store_scatter(srcbuf, [pos], sv, mask=m)
                plsc.store_scatter(dstbuf, [pos], dv - lo, mask=m)
                cnt_vec = cnt_vec + plsc.all_reduce_population_count(m)
            return cnt_vec
        cnt_vec = lax.fori_loop(0, _C // (4 * _L), filt_body,
                                jnp.zeros((_L,), jnp.int32))

        for t in range(2):
            tpos = cnt_vec + t * _L + iota16
            plsc.store_scatter(srcbuf, [tpos], jnp.zeros((_L,), jnp.int32))
            plsc.store_scatter(dstbuf, [tpos],
                               jnp.full((_L,), _R, jnp.int32))

        cnt = jnp.max(cnt_vec)
        tot = 8 * ((cnt + 15) // 8)

        sv0_ = srcbuf[pl.ds(0, _L)]
        for i in range(8):
            fetch_row(sv0_[i], rings[i], sems[i])

        def oct_loop(q, _):
            e = 8 * q
            svn = srcbuf[pl.ds(e + 8, _L)]
            svs = [svn[i] for i in range(8)]
            for i in range(8):
                pltpu.make_async_copy(bsh.at[pl.ds(0, _H)], rings[i],
                                      sems[i]).wait()
                accumulate(rings[i], e + i)
                @pl.when(e + 8 + i < tot)
                def _(i=i):
                    fetch_row(svs[i], rings[i], sems[i])
            return 0
        lax.fori_loop(0, tot // 8, oct_loop, 0)

    issue_chunk(0, srcchA, dstchA, semSA, semDA)

    def outer_body(cc, _):
        wait_chunk(srcchA, dstchA, semSA, semDA)
        issue_chunk(2 * cc + 1, srcchB, dstchB, semSB, semDB)
        process_chunk(srcchA, dstchA)

        wait_chunk(srcchB, dstchB, semSB, semDB)
        @pl.when(cc + 1 < _NCHUNK // 2)
        def _():
            issue_chunk(2 * cc + 2, srcchA, dstchA, semSA, semDA)
        process_chunk(srcchB, dstchB)
        return 0
    lax.fori_loop(0, _NCHUNK // 2, outer_body, 0)

    slab_rows = 64
    for s in range(_R // slab_rows):
        woff = (cid * _NPAD + lo + s * slab_rows) * _H
        pltpu.sync_copy(a2_hbm.at[pl.ds(woff, slab_rows * _H)], slab)

        def out_body(i, _):
            av = acc[pl.ds(s * slab_rows * _H + i * _L, _L)]
            sl = slab[pl.ds(i * _L, _L)]
            slab[pl.ds(i * _L, _L)] = jnp.where(av == jnp.inf, 0.0, sl - av)
            return 0
        lax.fori_loop(0, (slab_rows * _H) // _L, out_body, 0)
        pltpu.sync_copy(slab, out_hbm.at[pl.ds(woff, slab_rows * _H)])


@functools.partial(
    pl.kernel,
    mesh=plsc.VectorSubcoreMesh(core_axis_name="c", subcore_axis_name="s"),
    compiler_params=pltpu.CompilerParams(needs_layout_passes=False),
    out_type=jax.ShapeDtypeStruct((2 * _NPAD * _H,), jnp.float32),
    scratch_types=[
        pltpu.VMEM((_ACC_ROWS * _H,), jnp.float32),
        pltpu.VMEM((_C,), jnp.int32),
        pltpu.VMEM((_C,), jnp.int32),
        pltpu.VMEM((_C,), jnp.int32),
        pltpu.VMEM((_C,), jnp.int32),
        pltpu.VMEM((_FBUF,), jnp.int32),
        pltpu.VMEM((_FBUF,), jnp.int32),
        pltpu.VMEM((64 * _H,), jnp.float32),
        pltpu.VMEM((_H,), jnp.float32),
        pltpu.VMEM((_H,), jnp.float32),
        pltpu.VMEM((_H,), jnp.float32),
        pltpu.VMEM((_H,), jnp.float32),
        pltpu.VMEM((_H,), jnp.float32),
        pltpu.VMEM((_H,), jnp.float32),
        pltpu.VMEM((_H,), jnp.float32),
        pltpu.VMEM((_H,), jnp.float32),
        pltpu.VMEM_SHARED((_NPAD * _H,), jnp.float32),
        pltpu.SemaphoreType.DMA,
        pltpu.SemaphoreType.DMA,
        pltpu.SemaphoreType.DMA,
        pltpu.SemaphoreType.DMA,
        pltpu.SemaphoreType.DMA,
        pltpu.SemaphoreType.DMA,
        pltpu.SemaphoreType.DMA,
        pltpu.SemaphoreType.DMA,
        pltpu.SemaphoreType.DMA,
        pltpu.SemaphoreType.DMA,
        pltpu.SemaphoreType.DMA,
        pltpu.SemaphoreType.DMA,
    ],
)
def _seg_min(b2_hbm, src_hbm, dst_hbm, a2_hbm, out_hbm, *scratch):
    _seg_min_body(b2_hbm, src_hbm, dst_hbm, a2_hbm, out_hbm, *scratch)


def kernel(x, edge_index, W_theta, b_theta, W_phi, b_phi):
    a, b = _compute_ab(x, W_theta, b_theta, W_phi, b_phi)
    b_pad = jnp.pad(b, ((0, _NPAD - _N), (0, 0)))
    b2 = jnp.concatenate([b_pad[:, :_H], b_pad[:, _H:]], axis=0).reshape(-1)
    a_pad = jnp.pad(a, ((0, _NPAD - _N), (0, 0)))
    a2 = jnp.concatenate([a_pad[:, :_H], a_pad[:, _H:]], axis=0).reshape(-1)
    src = edge_index[0]
    dst = edge_index[1]
    out_flat = _seg_min(b2, src, dst, a2)
    o3 = out_flat.reshape(2, _NPAD, _H)
    return jnp.concatenate([o3[0, :_N], o3[1, :_N]], axis=1)
